# baseline (speedup 1.0000x reference)
import functools

import jax
import jax.numpy as jnp
from jax import lax
from jax.experimental import pallas as pl
from jax.experimental.pallas import tpu as pltpu
from jax.experimental.pallas import tpu_sc as plsc

_NEG = -1e30


def _hreduce(x, op):
    vals = [x[i] for i in range(16)]
    while len(vals) > 1:
        half = len(vals) // 2
        vals = [op(vals[i], vals[i + half]) for i in range(half)]
    return vals[0]


def _gates_body(feat_ref, ids_ref, wt_ref, b_ref, g_ref, off_ref, cnt_ref,
                *, nblocks, R, B, K, KP, D, N):
    i = pl.program_id(0)

    @pl.when(i == 0)
    def _init():
        cnt_ref[...] = jnp.zeros_like(cnt_ref)

    feat = feat_ref[...]
    g = lax.dot_general(wt_ref[...], feat, (((1,), (1,)), ((), ())),
                        preferred_element_type=jnp.float32)
    g_ref[...] = g + b_ref[:, 0:1]

    ids = ids_ref[0, 0, :]
    seg_iota = lax.broadcasted_iota(jnp.int32, (R, B), 1)
    row_iota = lax.broadcasted_iota(jnp.int32, (R, B), 0) + i * R
    p = ((ids[:, None] == seg_iota) & (row_iota < N)).astype(jnp.float32)
    cnt_ref[0, :B] += jnp.sum(p, axis=0)

    @pl.when(i == nblocks - 1)
    def _final():
        ext = jnp.concatenate(
            [cnt_ref[0, :B], jnp.zeros((128 - B,), jnp.float32)])
        incl = ext
        for sh in (1, 2, 4, 8, 16, 32, 64):
            incl = incl + jnp.concatenate(
                [jnp.zeros((sh,), jnp.float32), incl[:128 - sh]])
        off_ref[...] = (incl - ext).astype(jnp.int32)


def _sc_body(g_hbm, feat_hbm, off_hbm, out_hbm, off_v, gbuf, fbuf, obuf,
             gsem, fsem, *, N, N2, B, K, KP, D, CQ, NC, NW, SPW):
    wid = lax.axis_index("s") * NC + lax.axis_index("c")
    pltpu.sync_copy(off_hbm, off_v)
    lane = lax.iota(jnp.int32, 16)
    ntile = CQ // 16
    ccvecs = [cc * 16 + lane for cc in range(D // 16)]
    kvecs = [jnp.full((16,), k, jnp.int32) for k in range(K)]

    for s_local in range(SPW):
        s = wid * SPW + s_local
        offs = off_v[pl.ds(s, 16)]
        start = offs[0]
        end = offs[1]
        abase = (start // 16) * 16
        span = end - abase
        nch = lax.div(span + (CQ - 1), CQ)

        def g_issue(c, p):
            for k in range(K):
                pltpu.async_copy(
                    g_hbm.at[pl.ds(k * N2 + abase + c * CQ, CQ)],
                    gbuf.at[p, k], gsem.at[p])

        def g_wait(p):
            for k in range(K):
                pltpu.make_async_copy(g_hbm.at[pl.ds(0, CQ)], gbuf.at[p, k],
                                      gsem.at[p]).wait()

        def f_issue(c, p):
            fb2 = jnp.minimum(abase + c * CQ, N - CQ)
            pltpu.async_copy(feat_hbm.at[pl.ds(fb2, CQ), :],
                             fbuf.at[p], fsem.at[p])

        def f_wait(p):
            pltpu.make_async_copy(feat_hbm.at[pl.ds(0, CQ), :], fbuf.at[p],
                                  fsem.at[p]).wait()

        g_issue(0, 0)

        def pass_a(c, mk):
            g_issue(jnp.minimum(c + 1, nch - 1), lax.rem(c + 1, 2))
            p = lax.rem(c, 2)
            g_wait(p)
            pvec = jnp.full((16,), p, jnp.int32)
            cbase = abase + c * CQ

            def tile_a(j, mk):
                r = cbase + j * 16 + lane
                valid = (r >= start) & (r < end)
                idx = j * 16 + lane
                out = []
                for k in range(K):
                    g = plsc.load_gather(gbuf, [pvec, kvecs[k], idx])
                    out.append(jnp.maximum(
                        mk[k], jnp.where(valid, g, jnp.float32(_NEG))))
                return tuple(out)

            return lax.fori_loop(0, ntile, tile_a, mk)

        mk0 = tuple(jnp.full((16,), _NEG, jnp.float32) for _ in range(K))
        mk = lax.fori_loop(0, nch, pass_a, mk0)
        g_wait(lax.rem(nch, 2))
        ms = [_hreduce(mk[k], jnp.maximum) for k in range(K)]

        g_issue(0, 0)
        f_issue(0, 0)

        def pass_b(c, carry):
            acc, dacc = carry
            cnext = jnp.minimum(c + 1, nch - 1)
            pnext = lax.rem(c + 1, 2)
            g_issue(cnext, pnext)
            f_issue(cnext, pnext)
            p = lax.rem(c, 2)
            g_wait(p)
            f_wait(p)
            pvec = jnp.full((16,), p, jnp.int32)
            cbase = abase + c * CQ
            fd = cbase - jnp.minimum(cbase, N - CQ)

            def tile_b(j, carry):
                acc, dacc = carry
                r = cbase + j * 16 + lane
                valid = (r >= start) & (r < end)
                dacc = list(dacc)
                idx = j * 16 + lane
                for k in range(K):
                    g = plsc.load_gather(gbuf, [pvec, kvecs[k], idx])
                    e = jnp.where(valid, jnp.exp(g - ms[k]),
                                  jnp.float32(0.0))
                    dacc[k] = dacc[k] + e
                acc = list(acc)
                for r16 in range(16):
                    rr = cbase + j * 16 + r16
                    validf = jnp.where((rr >= start) & (rr < end),
                                       jnp.float32(1.0), jnp.float32(0.0))
                    row = jnp.minimum(fd + j * 16 + r16, CQ - 1)
                    rvec = jnp.full((16,), row, jnp.int32)
                    gidx = jnp.full((16,), j * 16 + r16, jnp.int32)
                    fv = [plsc.load_gather(fbuf, [pvec, rvec, ccvecs[cc]])
                          for cc in range(D // 16)]
                    for k in range(K):
                        gb = plsc.load_gather(gbuf, [pvec, kvecs[k], gidx])
                        ev = jnp.exp(jnp.minimum(gb - ms[k],
                                                 jnp.float32(0.0))) * validf
                        for cc in range(D // 16):
                            acc[k * (D // 16) + cc] = (
                                acc[k * (D // 16) + cc] + ev * fv[cc])
                return tuple(acc), tuple(dacc)

            return lax.fori_loop(0, ntile, tile_b, (acc, dacc))

        acc0 = tuple(jnp.zeros((16,), jnp.float32)
                     for _ in range(K * (D // 16)))
        dacc0 = tuple(jnp.zeros((16,), jnp.float32) for _ in range(K))
        acc, dacc = lax.fori_loop(0, nch, pass_b, (acc0, dacc0))
        g_wait(lax.rem(nch, 2))
        f_wait(lax.rem(nch, 2))

        for k in range(K):
            dk = _hreduce(dacc[k], jnp.add)
            dkv = jnp.full((16,), dk, jnp.float32)
            inv = jnp.where(dkv > 0,
                            jnp.full((16,), 1.0, jnp.float32) / dkv,
                            jnp.zeros((16,), jnp.float32))
            for cc in range(D // 16):
                obuf[k, pl.ds(cc * 16, 16)] = acc[k * (D // 16) + cc] * inv
        pltpu.sync_copy(obuf, out_hbm.at[s])


@jax.jit
def kernel(feat, segment_ids, W, b):
    N, D = feat.shape
    K = W.shape[0]
    B = 64
    KP = K
    CQ = 256

    R = 2048
    nblocks = -(-N // R)
    N2 = nblocks * R

    feat = feat.astype(jnp.float32)
    ids_pad = jnp.pad(segment_ids.astype(jnp.int32), (0, N2 - N),
                      constant_values=B)
    ids3 = ids_pad.reshape(nblocks, 1, R)
    wt = W.astype(jnp.float32)
    b8 = jnp.broadcast_to(b.astype(jnp.float32)[:, None], (KP, 8))

    body = functools.partial(_gates_body, nblocks=nblocks, R=R, B=B, K=K,
                             KP=KP, D=D, N=N)
    gates_i, offsets = pl.pallas_call(
        body,
        grid=(nblocks,),
        in_specs=[
            pl.BlockSpec((R, D), lambda i: (i, 0)),
            pl.BlockSpec((1, 1, R), lambda i: (i, 0, 0)),
            pl.BlockSpec((KP, D), lambda i: (0, 0)),
            pl.BlockSpec((KP, 8), lambda i: (0, 0)),
        ],
        out_specs=[
            pl.BlockSpec((KP, R), lambda i: (0, i)),
            pl.BlockSpec((128,), lambda i: (0,)),
        ],
        out_shape=[
            jax.ShapeDtypeStruct((KP, N2), jnp.float32),
            jax.ShapeDtypeStruct((128,), jnp.int32),
        ],
        scratch_shapes=[pltpu.VMEM((8, 128), jnp.float32)],
        compiler_params=pltpu.CompilerParams(
            dimension_semantics=("arbitrary",)),
    )(feat, ids3, wt, b8)

    info = plsc.get_sparse_core_info()
    NC, NS = info.num_cores, info.num_subcores
    NW = NC * NS
    SPW = B // NW

    mesh = plsc.VectorSubcoreMesh(core_axis_name="c", subcore_axis_name="s")
    sc_body = functools.partial(_sc_body, N=N, N2=N2, B=B, K=K, KP=KP, D=D,
                                CQ=CQ, NC=NC, NW=NW, SPW=SPW)
    sc = pl.kernel(
        sc_body,
        mesh=mesh,
        compiler_params=pltpu.CompilerParams(needs_layout_passes=False,
                                             use_tc_tiling_on_sc=False),
        out_type=jax.ShapeDtypeStruct((B, K, D), jnp.float32),
        scratch_types=[
            pltpu.VMEM((128,), jnp.int32),
            pltpu.VMEM((2, K, CQ), jnp.float32),
            pltpu.VMEM((2, CQ, D), jnp.float32),
            pltpu.VMEM((K, D), jnp.float32),
            pltpu.SemaphoreType.DMA((2,)),
            pltpu.SemaphoreType.DMA((2,)),
        ],
    )
    return sc(gates_i.reshape(KP * N2), feat, offsets)


# speedup vs baseline: 1.2298x; 1.2298x over previous
import functools

import jax
import jax.numpy as jnp
from jax import lax
from jax.experimental import pallas as pl
from jax.experimental.pallas import tpu as pltpu
from jax.experimental.pallas import tpu_sc as plsc

_NEG = -1e30


def _hreduce(x, op):
    vals = [x[i] for i in range(16)]
    while len(vals) > 1:
        half = len(vals) // 2
        vals = [op(vals[i], vals[i + half]) for i in range(half)]
    return vals[0]


def _gates_body(feat_ref, ids_ref, wt_ref, b_ref, g_ref, off_ref, cnt_ref,
                *, nblocks, R, B, K, KP, D, N):
    i = pl.program_id(0)

    @pl.when(i == 0)
    def _init():
        cnt_ref[...] = jnp.zeros_like(cnt_ref)

    feat = feat_ref[...]
    g = lax.dot_general(wt_ref[...], feat, (((1,), (1,)), ((), ())),
                        preferred_element_type=jnp.float32)
    g_ref[...] = g + b_ref[:, 0:1]

    ids = ids_ref[0, 0, :]
    seg_iota = lax.broadcasted_iota(jnp.int32, (R, B), 1)
    row_iota = lax.broadcasted_iota(jnp.int32, (R, B), 0) + i * R
    p = ((ids[:, None] == seg_iota) & (row_iota < N)).astype(jnp.float32)
    cnt_ref[0, :B] += jnp.sum(p, axis=0)

    @pl.when(i == nblocks - 1)
    def _final():
        ext = jnp.concatenate(
            [cnt_ref[0, :B], jnp.zeros((128 - B,), jnp.float32)])
        incl = ext
        for sh in (1, 2, 4, 8, 16, 32, 64):
            incl = incl + jnp.concatenate(
                [jnp.zeros((sh,), jnp.float32), incl[:128 - sh]])
        off_ref[...] = (incl - ext).astype(jnp.int32)


def _sc_body(g_hbm, feat_hbm, off_hbm, out_hbm, off_v, gbuf, fbuf, obuf,
             gsem, fsem, *, N, N2, B, K, KP, D, CQ, NC, NW, SPW):
    wid = lax.axis_index("s") * NC + lax.axis_index("c")
    pltpu.sync_copy(off_hbm, off_v)
    lane = lax.iota(jnp.int32, 16)
    ntile = CQ // 16
    ccvecs = [cc * 16 + lane for cc in range(D // 16)]
    kvecs = [jnp.full((16,), k, jnp.int32) for k in range(K)]

    for s_local in range(SPW):
        s = wid * SPW + s_local
        offs = off_v[pl.ds(s, 16)]
        start = offs[0]
        end = offs[1]
        abase = (start // 16) * 16
        span = end - abase
        nch = lax.div(span + (CQ - 1), CQ)

        def g_issue(c, p):
            for k in range(K):
                pltpu.async_copy(
                    g_hbm.at[pl.ds(k * N2 + abase + c * CQ, CQ)],
                    gbuf.at[p, k], gsem.at[p])

        def g_wait(p):
            for k in range(K):
                pltpu.make_async_copy(g_hbm.at[pl.ds(0, CQ)], gbuf.at[p, k],
                                      gsem.at[p]).wait()

        def f_issue(c, p):
            fb2 = jnp.minimum(abase + c * CQ, N - CQ)
            pltpu.async_copy(feat_hbm.at[pl.ds(fb2, CQ), :],
                             fbuf.at[p], fsem.at[p])

        def f_wait(p):
            pltpu.make_async_copy(feat_hbm.at[pl.ds(0, CQ), :], fbuf.at[p],
                                  fsem.at[p]).wait()

        g_issue(0, 0)

        def pass_a(c, mk):
            g_issue(jnp.minimum(c + 1, nch - 1), lax.rem(c + 1, 2))
            p = lax.rem(c, 2)
            g_wait(p)
            pvec = jnp.full((16,), p, jnp.int32)
            cbase = abase + c * CQ

            def tile_a(j, mk):
                r = cbase + j * 16 + lane
                valid = (r >= start) & (r < end)
                idx = j * 16 + lane
                out = []
                for k in range(K):
                    g = plsc.load_gather(gbuf, [pvec, kvecs[k], idx])
                    out.append(jnp.maximum(
                        mk[k], jnp.where(valid, g, jnp.float32(_NEG))))
                return tuple(out)

            return lax.fori_loop(0, ntile, tile_a, mk)

        mk0 = tuple(jnp.full((16,), _NEG, jnp.float32) for _ in range(K))
        mk = lax.fori_loop(0, nch, pass_a, mk0)
        g_wait(lax.rem(nch, 2))
        ms = [_hreduce(mk[k], jnp.maximum) for k in range(K)]

        g_issue(0, 0)
        f_issue(0, 0)

        def pass_b(c, carry):
            acc, dacc = carry
            cnext = jnp.minimum(c + 1, nch - 1)
            pnext = lax.rem(c + 1, 2)
            g_issue(cnext, pnext)
            f_issue(cnext, pnext)
            p = lax.rem(c, 2)
            g_wait(p)
            f_wait(p)
            pvec = jnp.full((16,), p, jnp.int32)
            cbase = abase + c * CQ
            fd = cbase - jnp.minimum(cbase, N - CQ)

            def tile_b(j, carry):
                acc, dacc = carry
                r = cbase + j * 16 + lane
                valid = (r >= start) & (r < end)
                dacc = list(dacc)
                idx = j * 16 + lane
                es = []
                for k in range(K):
                    g = plsc.load_gather(gbuf, [pvec, kvecs[k], idx])
                    e = jnp.where(valid, jnp.exp(g - ms[k]),
                                  jnp.float32(0.0))
                    dacc[k] = dacc[k] + e
                    es.append(e)
                acc = list(acc)
                for r16 in range(16):
                    row = jnp.minimum(fd + j * 16 + r16, CQ - 1)
                    rvec = jnp.full((16,), row, jnp.int32)
                    fv = [plsc.load_gather(fbuf, [pvec, rvec, ccvecs[cc]])
                          for cc in range(D // 16)]
                    for k in range(K):
                        ev = jnp.full((16,), es[k][r16], jnp.float32)
                        for cc in range(D // 16):
                            acc[k * (D // 16) + cc] = (
                                acc[k * (D // 16) + cc] + ev * fv[cc])
                return tuple(acc), tuple(dacc)

            return lax.fori_loop(0, ntile, tile_b, (acc, dacc))

        acc0 = tuple(jnp.zeros((16,), jnp.float32)
                     for _ in range(K * (D // 16)))
        dacc0 = tuple(jnp.zeros((16,), jnp.float32) for _ in range(K))
        acc, dacc = lax.fori_loop(0, nch, pass_b, (acc0, dacc0))
        g_wait(lax.rem(nch, 2))
        f_wait(lax.rem(nch, 2))

        for k in range(K):
            dk = _hreduce(dacc[k], jnp.add)
            dkv = jnp.full((16,), dk, jnp.float32)
            inv = jnp.where(dkv > 0,
                            jnp.full((16,), 1.0, jnp.float32) / dkv,
                            jnp.zeros((16,), jnp.float32))
            for cc in range(D // 16):
                obuf[k, pl.ds(cc * 16, 16)] = acc[k * (D // 16) + cc] * inv
        pltpu.sync_copy(obuf, out_hbm.at[s])


@jax.jit
def kernel(feat, segment_ids, W, b):
    N, D = feat.shape
    K = W.shape[0]
    B = 64
    KP = K
    CQ = 256

    R = 2048
    nblocks = -(-N // R)
    N2 = nblocks * R

    feat = feat.astype(jnp.float32)
    ids_pad = jnp.pad(segment_ids.astype(jnp.int32), (0, N2 - N),
                      constant_values=B)
    ids3 = ids_pad.reshape(nblocks, 1, R)
    wt = W.astype(jnp.float32)
    b8 = jnp.broadcast_to(b.astype(jnp.float32)[:, None], (KP, 8))

    body = functools.partial(_gates_body, nblocks=nblocks, R=R, B=B, K=K,
                             KP=KP, D=D, N=N)
    gates_i, offsets = pl.pallas_call(
        body,
        grid=(nblocks,),
        in_specs=[
            pl.BlockSpec((R, D), lambda i: (i, 0)),
            pl.BlockSpec((1, 1, R), lambda i: (i, 0, 0)),
            pl.BlockSpec((KP, D), lambda i: (0, 0)),
            pl.BlockSpec((KP, 8), lambda i: (0, 0)),
        ],
        out_specs=[
            pl.BlockSpec((KP, R), lambda i: (0, i)),
            pl.BlockSpec((128,), lambda i: (0,)),
        ],
        out_shape=[
            jax.ShapeDtypeStruct((KP, N2), jnp.float32),
            jax.ShapeDtypeStruct((128,), jnp.int32),
        ],
        scratch_shapes=[pltpu.VMEM((8, 128), jnp.float32)],
        compiler_params=pltpu.CompilerParams(
            dimension_semantics=("arbitrary",)),
    )(feat, ids3, wt, b8)

    info = plsc.get_sparse_core_info()
    NC, NS = info.num_cores, info.num_subcores
    NW = NC * NS
    SPW = B // NW

    mesh = plsc.VectorSubcoreMesh(core_axis_name="c", subcore_axis_name="s")
    sc_body = functools.partial(_sc_body, N=N, N2=N2, B=B, K=K, KP=KP, D=D,
                                CQ=CQ, NC=NC, NW=NW, SPW=SPW)
    sc = pl.kernel(
        sc_body,
        mesh=mesh,
        compiler_params=pltpu.CompilerParams(needs_layout_passes=False,
                                             use_tc_tiling_on_sc=False),
        out_type=jax.ShapeDtypeStruct((B, K, D), jnp.float32),
        scratch_types=[
            pltpu.VMEM((128,), jnp.int32),
            pltpu.VMEM((2, K, CQ), jnp.float32),
            pltpu.VMEM((2, CQ, D), jnp.float32),
            pltpu.VMEM((K, D), jnp.float32),
            pltpu.SemaphoreType.DMA((2,)),
            pltpu.SemaphoreType.DMA((2,)),
        ],
    )
    return sc(gates_i.reshape(KP * N2), feat, offsets)
